# Initial kernel scaffold; baseline (speedup 1.0000x reference)
import jax
import jax.numpy as jnp
from jax.experimental import pallas as pl


def kernel(X, edge_index, W0, b0, W1, b1):
    raise NotImplementedError("write your pallas kernel here")


# speedup vs baseline: 19.3825x; 19.3825x over previous
import functools

import jax
import jax.numpy as jnp
from jax import lax
from jax.experimental import pallas as pl
from jax.experimental.pallas import tpu as pltpu
from jax.experimental.pallas import tpu_sc as plsc

N_NODES = 10000
N_EDGES = 320000
NC = 2
NS = 16
NW = NC * NS
EPT = N_EDGES // NW
CHUNK = 80
NCH = EPT // CHUNK
ROW_CHUNKS = N_NODES // CHUNK


def _mesh():
    return plsc.VectorSubcoreMesh(
        core_axis_name="c", subcore_axis_name="s", num_cores=NC, num_subcores=NS
    )


def _row_span(s):
    lo = 8 * s - jnp.maximum(s - 13, 0)
    n = jnp.where(s >= 13, 7, 8)
    return lo, n


_SC_PARAMS = pltpu.CompilerParams(use_tc_tiling_on_sc=False)


@functools.partial(
    pl.kernel,
    mesh=_mesh(),
    compiler_params=_SC_PARAMS,
    out_type=jax.ShapeDtypeStruct((NC * N_NODES,), jnp.float32),
    scratch_types=[
        pltpu.VMEM((NCH, CHUNK), jnp.int32),
        pltpu.VMEM((CHUNK,), jnp.float32),
        pltpu.VMEM((CHUNK,), jnp.float32),
        pltpu.VMEM_SHARED((N_NODES,), jnp.float32),
    ],
)
def _deg_kernel(dst_hbm, out_hbm, dstv, ones, zbuf, acc):
    c = lax.axis_index("c")
    s = lax.axis_index("s")
    wid = s * NC + c
    pltpu.sync_copy(dst_hbm.at[wid], dstv)
    for i in range(CHUNK // 16):
        ones[pl.ds(16 * i, 16)] = jnp.ones((16,), jnp.float32)
        zbuf[pl.ds(16 * i, 16)] = jnp.zeros((16,), jnp.float32)
    lo, nz = _row_span(s)

    def zero_chunk(i, carry):
        pltpu.sync_copy(zbuf, acc.at[pl.ds((lo + i) * CHUNK, CHUNK)])
        return carry

    lax.fori_loop(0, nz, zero_chunk, 0)
    plsc.subcore_barrier()

    def edge_chunk(j, carry):
        pltpu.sync_copy(ones, acc.at[dstv.at[j]], add=True)
        return carry

    lax.fori_loop(0, NCH, edge_chunk, 0)
    plsc.subcore_barrier()

    def out_chunk(i, carry):
        off = (lo + i) * CHUNK
        pltpu.sync_copy(acc.at[pl.ds(off, CHUNK)], zbuf)
        pltpu.sync_copy(zbuf, out_hbm.at[pl.ds(c * N_NODES + off, CHUNK)])
        return carry

    lax.fori_loop(0, nz, out_chunk, 0)


def _make_smooth(d):
    @functools.partial(
        pl.kernel,
        mesh=_mesh(),
        compiler_params=_SC_PARAMS,
        out_type=jax.ShapeDtypeStruct((NC, N_NODES, d), jnp.float32),
        scratch_types=[
            pltpu.VMEM((NCH, CHUNK), jnp.int32),
            pltpu.VMEM((NCH, CHUNK), jnp.int32),
            pltpu.VMEM((CHUNK, d), jnp.float32),
            pltpu.VMEM((CHUNK, d), jnp.float32),
            pltpu.VMEM((CHUNK, d), jnp.float32),
            pltpu.VMEM_SHARED((N_NODES, d), jnp.float32),
            pltpu.SemaphoreType.DMA,
            pltpu.SemaphoreType.DMA,
        ],
    )
    def smooth(src_hbm, dst_hbm, g_hbm, out_hbm, srcv, dstv, rows0, rows1, zbuf,
               acc, sem0, sem1):
        c = lax.axis_index("c")
        s = lax.axis_index("s")
        wid = s * NC + c
        pltpu.sync_copy(src_hbm.at[wid], srcv)
        pltpu.sync_copy(dst_hbm.at[wid], dstv)

        def zero_row(r, carry):
            for j in range(d // 16):
                zbuf[r, pl.ds(16 * j, 16)] = jnp.zeros((16,), jnp.float32)
            return carry

        lax.fori_loop(0, CHUNK, zero_row, 0)
        lo, nz = _row_span(s)

        def zero_chunk(i, carry):
            pltpu.sync_copy(zbuf, acc.at[pl.ds((lo + i) * CHUNK, CHUNK)])
            return carry

        lax.fori_loop(0, nz, zero_chunk, 0)
        plsc.subcore_barrier()

        def edge_pair(jj, carry):
            j0 = 2 * jj
            cp0 = pltpu.async_copy(g_hbm.at[srcv.at[j0]], rows0, sem0)
            cp1 = pltpu.async_copy(g_hbm.at[srcv.at[j0 + 1]], rows1, sem1)
            cp0.wait()
            pltpu.sync_copy(rows0, acc.at[dstv.at[j0]], add=True)
            cp1.wait()
            pltpu.sync_copy(rows1, acc.at[dstv.at[j0 + 1]], add=True)
            return carry

        lax.fori_loop(0, NCH // 2, edge_pair, 0)
        cp = pltpu.async_copy(g_hbm.at[srcv.at[NCH - 1]], rows0, sem0)
        cp.wait()
        pltpu.sync_copy(rows0, acc.at[dstv.at[NCH - 1]], add=True)
        plsc.subcore_barrier()

        def out_chunk(i, carry):
            off = (lo + i) * CHUNK
            pltpu.sync_copy(acc.at[pl.ds(off, CHUNK)], rows0)
            pltpu.sync_copy(rows0, out_hbm.at[c, pl.ds(off, CHUNK)])
            return carry

        lax.fori_loop(0, nz, out_chunk, 0)

    return smooth


_smooth64 = _make_smooth(64)


def _tc_layer0(deg, x, w0, b0):
    def body(deg_ref, x_ref, w_ref, b_ref, norm_ref, g0a_ref, g0b_ref):
        dsum = deg_ref[0, :] + deg_ref[1, :] + 1.0
        norm = lax.rsqrt(dsum)
        norm_ref[...] = norm[:, None]
        h = jnp.dot(x_ref[...], w_ref[...], preferred_element_type=jnp.float32)
        h = h + b_ref[...]
        g0 = h * norm[:, None]
        g0a_ref[...] = g0[:, :64]
        g0b_ref[...] = g0[:, 64:]

    return pl.pallas_call(
        body,
        out_shape=(
            jax.ShapeDtypeStruct((N_NODES, 1), jnp.float32),
            jax.ShapeDtypeStruct((N_NODES, 64), jnp.float32),
            jax.ShapeDtypeStruct((N_NODES, 64), jnp.float32),
        ),
    )(deg, x, w0, b0)


def _tc_layer1(pa, pb, g0a, g0b, norm, w1, b1):
    def body(pa_ref, pb_ref, g0a_ref, g0b_ref, norm_ref, w_ref, b_ref, g1_ref):
        sma = (pa_ref[0] + pa_ref[1] + g0a_ref[...]) * norm_ref[...]
        smb = (pb_ref[0] + pb_ref[1] + g0b_ref[...]) * norm_ref[...]
        h1 = jnp.maximum(jnp.concatenate([sma, smb], axis=1), 0.0)
        o = jnp.dot(h1, w_ref[...], preferred_element_type=jnp.float32)
        o = o + b_ref[...]
        g1_ref[...] = o * norm_ref[...]

    return pl.pallas_call(
        body,
        out_shape=jax.ShapeDtypeStruct((N_NODES, w1.shape[1]), jnp.float32),
    )(pa, pb, g0a, g0b, norm, w1, b1)


def _tc_final(q, g1, norm):
    def body(q_ref, g1_ref, norm_ref, o_ref):
        o_ref[...] = (q_ref[0] + q_ref[1] + g1_ref[...]) * norm_ref[...]

    return pl.pallas_call(
        body,
        out_shape=jax.ShapeDtypeStruct(g1.shape, jnp.float32),
    )(q, g1, norm)


def kernel(X, edge_index, W0, b0, W1, b1):
    src = edge_index[0].astype(jnp.int32).reshape(NW, NCH, CHUNK)
    dst = edge_index[1].astype(jnp.int32).reshape(NW, NCH, CHUNK)
    deg = _deg_kernel(dst).reshape(NC, N_NODES)
    norm, g0a, g0b = _tc_layer0(deg, X, W0, b0.reshape(1, -1))
    pa = _smooth64(src, dst, g0a)
    pb = _smooth64(src, dst, g0b)
    g1 = _tc_layer1(pa, pb, g0a, g0b, norm, W1, b1.reshape(1, -1))
    q = _smooth64(src, dst, g1)
    return _tc_final(q, g1, norm)


# speedup vs baseline: 30.5509x; 1.5762x over previous
import functools

import jax
import jax.numpy as jnp
from jax import lax
from jax.experimental import pallas as pl
from jax.experimental.pallas import tpu as pltpu
from jax.experimental.pallas import tpu_sc as plsc

N_NODES = 10000
N_EDGES = 320000
NC = 2
NS = 16
CHUNK = 80
SLAB_ROWS = N_EDGES // NS // CHUNK
BC = 5
BUF_E = BC * CHUNK
NBC_FULL = SLAB_ROWS // BC
NBC_HALF = SLAB_ROWS // NC // BC
ZROWS = 104

_SC_PARAMS = pltpu.CompilerParams(use_tc_tiling_on_sc=False)


def _mesh():
    return plsc.VectorSubcoreMesh(
        core_axis_name="c", subcore_axis_name="s", num_cores=NC, num_subcores=NS
    )


def _zero_vmem(buf, rows, d):
    def zero_row(r, carry):
        for j in range(d // 16):
            buf[r, pl.ds(16 * j, 16)] = jnp.zeros((16,), jnp.float32)
        return carry

    lax.fori_loop(0, rows, zero_row, 0)


def _node_span(s):
    return 624 * s


def _zero_acc(acc, zbuf, s):
    base = _node_span(s)
    for i in range(6):
        pltpu.sync_copy(zbuf, acc.at[pl.ds(base + ZROWS * i, ZROWS)])

    @pl.when(s == 15)
    def _():
        pltpu.sync_copy(zbuf.at[pl.ds(0, 16)], acc.at[pl.ds(9984, 16)])


def _copy_out(acc, bounce, out_slice, s):
    base = _node_span(s)
    for i in range(6):
        sl = pl.ds(base + ZROWS * i, ZROWS)
        pltpu.sync_copy(acc.at[sl], bounce.at[pl.ds(0, ZROWS)])
        pltpu.sync_copy(bounce.at[pl.ds(0, ZROWS)], out_slice.at[sl])

    @pl.when(s == 15)
    def _():
        pltpu.sync_copy(acc.at[pl.ds(9984, 16)], bounce.at[pl.ds(0, 16)])
        pltpu.sync_copy(bounce.at[pl.ds(0, 16)], out_slice.at[pl.ds(9984, 16)])


def _edge_pipeline(src_slab, dst_slab, g_src, acc, isrc, idst, isems, bufs,
                   gsems, ssems, row_lo, n_rows):
    GR = n_rows // 5
    NBC = n_rows // BC
    BPG = GR // BC
    gpend = {0: None, 1: None}
    spend = {0: None, 1: None}
    ipend = {0: None, 1: None}

    def issue_idx(grp):
        st = grp % 2
        sl = pl.ds(row_lo + grp * GR, GR)
        return [
            pltpu.async_copy(src_slab.at[sl], isrc[st], isems[st]),
            pltpu.async_copy(dst_slab.at[sl], idst[st], isems[st]),
        ]

    def issue_gathers(k, b):
        st = (k // BPG) % 2
        lr = (k % BPG) * BC
        return [
            pltpu.async_copy(
                g_src.at[isrc[st].at[lr + i]],
                bufs[b].at[pl.ds(CHUNK * i, CHUNK)],
                gsems[b],
            )
            for i in range(BC)
        ]

    def issue_scatters(k, b):
        st = (k // BPG) % 2
        lr = (k % BPG) * BC
        return [
            pltpu.async_copy(
                bufs[b].at[pl.ds(CHUNK * i, CHUNK)],
                acc.at[idst[st].at[lr + i]],
                ssems[b],
                add=True,
            )
            for i in range(BC)
        ]

    ipend[0] = issue_idx(0)
    for k in range(NBC):
        b = k % 2
        grp = k // BPG
        if k % BPG == 0 and ipend[grp % 2] is not None:
            for d_ in ipend[grp % 2]:
                d_.wait()
            ipend[grp % 2] = None
        if spend[b] is not None:
            for d_ in spend[b]:
                d_.wait()
            spend[b] = None
        gpend[b] = issue_gathers(k, b)
        if k % BPG == 2 and grp + 1 < 5:
            ipend[(grp + 1) % 2] = issue_idx(grp + 1)
        b1 = 1 - b
        if k >= 1 and gpend[b1] is not None:
            for d_ in gpend[b1]:
                d_.wait()
            gpend[b1] = None
            spend[b1] = issue_scatters(k - 1, b1)
    b_last = (NBC - 1) % 2
    for d_ in gpend[b_last]:
        d_.wait()
    spend[b_last] = issue_scatters(NBC - 1, b_last)
    for b in (0, 1):
        if spend[b] is not None:
            for d_ in spend[b]:
                d_.wait()


@functools.partial(
    pl.kernel,
    mesh=_mesh(),
    compiler_params=_SC_PARAMS,
    out_type=jax.ShapeDtypeStruct((NC * N_NODES,), jnp.float32),
    scratch_types=[
        pltpu.VMEM((SLAB_ROWS // NC, CHUNK), jnp.int32),
        pltpu.VMEM((CHUNK,), jnp.float32),
        pltpu.VMEM((ZROWS,), jnp.float32),
        pltpu.VMEM_SHARED((N_NODES,), jnp.float32),
        pltpu.SemaphoreType.DMA,
    ],
)
def _deg_kernel(dst_hbm, out_hbm, dstv, ones, zbuf, acc, dsem):
    c = lax.axis_index("c")
    s = lax.axis_index("s")
    nch = SLAB_ROWS // NC
    pltpu.sync_copy(dst_hbm.at[s, pl.ds(c * nch, nch)], dstv)
    for i in range(CHUNK // 16):
        ones[pl.ds(16 * i, 16)] = jnp.ones((16,), jnp.float32)
    for i in range(6):
        zbuf[pl.ds(16 * i, 16)] = jnp.zeros((16,), jnp.float32)
    zbuf[pl.ds(88, 16)] = jnp.zeros((16,), jnp.float32)
    base = 624 * s
    for i in range(6):
        pltpu.sync_copy(zbuf, acc.at[pl.ds(base + ZROWS * i, ZROWS)])

    @pl.when(s == 15)
    def _():
        pltpu.sync_copy(zbuf.at[pl.ds(0, 16)], acc.at[pl.ds(9984, 16)])

    plsc.subcore_barrier()

    def fire(j, carry):
        pltpu.async_copy(ones, acc.at[dstv.at[j]], dsem, add=True)
        return carry

    lax.fori_loop(0, nch, fire, 0)

    def drain(j, carry):
        pltpu.make_async_copy(ones, acc.at[dstv.at[0]], dsem).wait()
        return carry

    lax.fori_loop(0, nch, drain, 0)
    plsc.subcore_barrier()
    for i in range(6):
        sl = pl.ds(base + ZROWS * i, ZROWS)
        pltpu.sync_copy(acc.at[sl], zbuf)
        pltpu.sync_copy(zbuf, out_hbm.at[pl.ds(c * N_NODES + base + ZROWS * i, ZROWS)])

    @pl.when(s == 15)
    def _():
        pltpu.sync_copy(acc.at[pl.ds(9984, 16)], zbuf.at[pl.ds(0, 16)])
        pltpu.sync_copy(zbuf.at[pl.ds(0, 16)], out_hbm.at[pl.ds(c * N_NODES + 9984, 16)])


@functools.partial(
    pl.kernel,
    mesh=_mesh(),
    compiler_params=_SC_PARAMS,
    out_type=jax.ShapeDtypeStruct((NC, N_NODES, 64), jnp.float32),
    scratch_types=[
        pltpu.VMEM((SLAB_ROWS // 5, CHUNK), jnp.int32),
        pltpu.VMEM((SLAB_ROWS // 5, CHUNK), jnp.int32),
        pltpu.VMEM((SLAB_ROWS // 5, CHUNK), jnp.int32),
        pltpu.VMEM((SLAB_ROWS // 5, CHUNK), jnp.int32),
        pltpu.VMEM((BUF_E, 64), jnp.float32),
        pltpu.VMEM((BUF_E, 64), jnp.float32),
        pltpu.VMEM((ZROWS, 64), jnp.float32),
        pltpu.VMEM_SHARED((N_NODES, 64), jnp.float32),
        pltpu.SemaphoreType.DMA,
        pltpu.SemaphoreType.DMA,
        pltpu.SemaphoreType.DMA,
        pltpu.SemaphoreType.DMA,
        pltpu.SemaphoreType.DMA,
        pltpu.SemaphoreType.DMA,
    ],
)
def _smooth_full(src_hbm, dst_hbm, g_hbm, out_hbm, isrc0, isrc1, idst0, idst1,
                 buf0, buf1, zbuf, acc, is0, is1, gs0, gs1, ss0, ss1):
    c = lax.axis_index("c")
    s = lax.axis_index("s")
    _zero_vmem(zbuf, ZROWS, 64)
    _zero_acc(acc, zbuf, s)
    plsc.subcore_barrier()
    _edge_pipeline(
        src_hbm.at[s], dst_hbm.at[s], g_hbm.at[c], acc,
        (isrc0, isrc1), (idst0, idst1), (is0, is1),
        (buf0, buf1), (gs0, gs1), (ss0, ss1),
        0, SLAB_ROWS,
    )
    plsc.subcore_barrier()
    _copy_out(acc, buf0, out_hbm.at[c], s)


@functools.partial(
    pl.kernel,
    mesh=_mesh(),
    compiler_params=_SC_PARAMS,
    out_type=jax.ShapeDtypeStruct((NC, N_NODES, 64), jnp.float32),
    scratch_types=[
        pltpu.VMEM((SLAB_ROWS // NC // 5, CHUNK), jnp.int32),
        pltpu.VMEM((SLAB_ROWS // NC // 5, CHUNK), jnp.int32),
        pltpu.VMEM((SLAB_ROWS // NC // 5, CHUNK), jnp.int32),
        pltpu.VMEM((SLAB_ROWS // NC // 5, CHUNK), jnp.int32),
        pltpu.VMEM((BUF_E, 64), jnp.float32),
        pltpu.VMEM((BUF_E, 64), jnp.float32),
        pltpu.VMEM((ZROWS, 64), jnp.float32),
        pltpu.VMEM_SHARED((N_NODES, 64), jnp.float32),
        pltpu.SemaphoreType.DMA,
        pltpu.SemaphoreType.DMA,
        pltpu.SemaphoreType.DMA,
        pltpu.SemaphoreType.DMA,
        pltpu.SemaphoreType.DMA,
        pltpu.SemaphoreType.DMA,
    ],
)
def _smooth_half(src_hbm, dst_hbm, g_hbm, out_hbm, isrc0, isrc1, idst0, idst1,
                 buf0, buf1, zbuf, acc, is0, is1, gs0, gs1, ss0, ss1):
    c = lax.axis_index("c")
    s = lax.axis_index("s")
    nch = SLAB_ROWS // NC
    _zero_vmem(zbuf, ZROWS, 64)
    _zero_acc(acc, zbuf, s)
    plsc.subcore_barrier()
    _edge_pipeline(
        src_hbm.at[s], dst_hbm.at[s], g_hbm, acc,
        (isrc0, isrc1), (idst0, idst1), (is0, is1),
        (buf0, buf1), (gs0, gs1), (ss0, ss1),
        c * nch, nch,
    )
    plsc.subcore_barrier()
    _copy_out(acc, buf0, out_hbm.at[c], s)


def _tc_layer0(deg, x, w0, b0):
    def body(deg_ref, x_ref, w_ref, b_ref, norm_ref, g0_ref):
        dsum = deg_ref[0, :] + deg_ref[1, :] + 1.0
        norm = lax.rsqrt(dsum)
        norm_ref[...] = norm[:, None]
        h = jnp.dot(x_ref[...], w_ref[...], preferred_element_type=jnp.float32)
        h = h + b_ref[...]
        g0 = h * norm[:, None]
        g0_ref[0] = g0[:, :64]
        g0_ref[1] = g0[:, 64:]

    return pl.pallas_call(
        body,
        out_shape=(
            jax.ShapeDtypeStruct((N_NODES, 1), jnp.float32),
            jax.ShapeDtypeStruct((NC, N_NODES, 64), jnp.float32),
        ),
    )(deg, x, w0, b0)


def _tc_layer1(p, g0, norm, w1, b1):
    def body(p_ref, g0_ref, norm_ref, w_ref, b_ref, g1_ref):
        sm = jnp.concatenate(
            [p_ref[0] + g0_ref[0], p_ref[1] + g0_ref[1]], axis=1
        ) * norm_ref[...]
        h1 = jnp.maximum(sm, 0.0)
        o = jnp.dot(h1, w_ref[...], preferred_element_type=jnp.float32)
        o = o + b_ref[...]
        g1_ref[...] = o * norm_ref[...]

    return pl.pallas_call(
        body,
        out_shape=jax.ShapeDtypeStruct((N_NODES, w1.shape[1]), jnp.float32),
    )(p, g0, norm, w1, b1)


def _tc_final(q, g1, norm):
    def body(q_ref, g1_ref, norm_ref, o_ref):
        o_ref[...] = (q_ref[0] + q_ref[1] + g1_ref[...]) * norm_ref[...]

    return pl.pallas_call(
        body,
        out_shape=jax.ShapeDtypeStruct(g1.shape, jnp.float32),
    )(q, g1, norm)


def kernel(X, edge_index, W0, b0, W1, b1):
    src = edge_index[0].astype(jnp.int32).reshape(NS, SLAB_ROWS, CHUNK)
    dst = edge_index[1].astype(jnp.int32).reshape(NS, SLAB_ROWS, CHUNK)
    deg = _deg_kernel(dst).reshape(NC, N_NODES)
    norm, g0 = _tc_layer0(deg, X, W0, b0.reshape(1, -1))
    p = _smooth_full(src, dst, g0)
    g1 = _tc_layer1(p, g0, norm, W1, b1.reshape(1, -1))
    q = _smooth_half(src, dst, g1)
    return _tc_final(q, g1, norm)


# speedup vs baseline: 30.9913x; 1.0144x over previous
import functools

import jax
import jax.numpy as jnp
from jax import lax
from jax.experimental import pallas as pl
from jax.experimental.pallas import tpu as pltpu
from jax.experimental.pallas import tpu_sc as plsc

N_NODES = 10000
N_EDGES = 320000
NC = 2
NS = 16
CHUNK = 80
SLAB_ROWS = N_EDGES // NS // CHUNK
BC = 5
BUF_E = BC * CHUNK
NBC_FULL = SLAB_ROWS // BC
NBC_HALF = SLAB_ROWS // NC // BC
ZROWS = 104

_SC_PARAMS = pltpu.CompilerParams(use_tc_tiling_on_sc=False)


def _mesh():
    return plsc.VectorSubcoreMesh(
        core_axis_name="c", subcore_axis_name="s", num_cores=NC, num_subcores=NS
    )


def _zero_vmem(buf, rows, d):
    def zero_row(r, carry):
        for j in range(d // 16):
            buf[r, pl.ds(16 * j, 16)] = jnp.zeros((16,), jnp.float32)
        return carry

    lax.fori_loop(0, rows, zero_row, 0)


def _node_span(s):
    return 624 * s


def _zero_acc(acc, zbuf, s):
    base = _node_span(s)
    for i in range(6):
        pltpu.sync_copy(zbuf, acc.at[pl.ds(base + ZROWS * i, ZROWS)])

    @pl.when(s == 15)
    def _():
        pltpu.sync_copy(zbuf.at[pl.ds(0, 16)], acc.at[pl.ds(9984, 16)])


def _copy_out(acc, bounce, out_slice, s):
    base = _node_span(s)
    for i in range(6):
        sl = pl.ds(base + ZROWS * i, ZROWS)
        pltpu.sync_copy(acc.at[sl], bounce.at[pl.ds(0, ZROWS)])
        pltpu.sync_copy(bounce.at[pl.ds(0, ZROWS)], out_slice.at[sl])

    @pl.when(s == 15)
    def _():
        pltpu.sync_copy(acc.at[pl.ds(9984, 16)], bounce.at[pl.ds(0, 16)])
        pltpu.sync_copy(bounce.at[pl.ds(0, 16)], out_slice.at[pl.ds(9984, 16)])


def _edge_pipeline(src_slab, dst_slab, g_src, acc, isrc, idst, isems, bufs,
                   gsems, ssems, row_lo, n_rows):
    GR = n_rows // 5
    GE = GR * CHUNK
    NBC = n_rows // BC
    BPG = GR // BC
    gpend = {0: None, 1: None}
    spend = {0: None, 1: None}
    ipend = {0: None, 1: None}

    def issue_idx(grp):
        st = grp % 2
        sl = pl.ds(row_lo * CHUNK + grp * GE, GE)
        return [
            pltpu.async_copy(src_slab.at[sl], isrc[st], isems[st]),
            pltpu.async_copy(dst_slab.at[sl], idst[st], isems[st]),
        ]

    def issue_gathers(k, b):
        st = (k // BPG) % 2
        lo = (k % BPG) * BUF_E
        return [
            pltpu.async_copy(
                g_src.at[isrc[st].at[pl.ds(lo, BUF_E)]],
                bufs[b],
                gsems[b],
            )
        ]

    def issue_scatters(k, b):
        st = (k // BPG) % 2
        lo = (k % BPG) * BUF_E
        return [
            pltpu.async_copy(
                bufs[b],
                acc.at[idst[st].at[pl.ds(lo, BUF_E)]],
                ssems[b],
                add=True,
            )
        ]

    ipend[0] = issue_idx(0)
    for k in range(NBC):
        b = k % 2
        grp = k // BPG
        if k % BPG == 0 and ipend[grp % 2] is not None:
            for d_ in ipend[grp % 2]:
                d_.wait()
            ipend[grp % 2] = None
        if spend[b] is not None:
            for d_ in spend[b]:
                d_.wait()
            spend[b] = None
        gpend[b] = issue_gathers(k, b)
        if k % BPG == 2 and grp + 1 < 5:
            ipend[(grp + 1) % 2] = issue_idx(grp + 1)
        b1 = 1 - b
        if k >= 1 and gpend[b1] is not None:
            for d_ in gpend[b1]:
                d_.wait()
            gpend[b1] = None
            spend[b1] = issue_scatters(k - 1, b1)
    b_last = (NBC - 1) % 2
    for d_ in gpend[b_last]:
        d_.wait()
    spend[b_last] = issue_scatters(NBC - 1, b_last)
    for b in (0, 1):
        if spend[b] is not None:
            for d_ in spend[b]:
                d_.wait()


@functools.partial(
    pl.kernel,
    mesh=_mesh(),
    compiler_params=_SC_PARAMS,
    out_type=jax.ShapeDtypeStruct((NC * N_NODES,), jnp.float32),
    scratch_types=[
        pltpu.VMEM((SLAB_ROWS // NC * CHUNK,), jnp.int32),
        pltpu.VMEM((BUF_E,), jnp.float32),
        pltpu.VMEM((ZROWS,), jnp.float32),
        pltpu.VMEM_SHARED((N_NODES,), jnp.float32),
        pltpu.SemaphoreType.DMA,
    ],
)
def _deg_kernel(dst_hbm, out_hbm, dstv, ones, zbuf, acc, dsem):
    c = lax.axis_index("c")
    s = lax.axis_index("s")
    ne = SLAB_ROWS // NC * CHUNK
    pltpu.sync_copy(dst_hbm.at[s, pl.ds(c * ne, ne)], dstv)
    for i in range(BUF_E // 16):
        ones[pl.ds(16 * i, 16)] = jnp.ones((16,), jnp.float32)
    for i in range(6):
        zbuf[pl.ds(16 * i, 16)] = jnp.zeros((16,), jnp.float32)
    zbuf[pl.ds(88, 16)] = jnp.zeros((16,), jnp.float32)
    base = 624 * s
    for i in range(6):
        pltpu.sync_copy(zbuf, acc.at[pl.ds(base + ZROWS * i, ZROWS)])

    @pl.when(s == 15)
    def _():
        pltpu.sync_copy(zbuf.at[pl.ds(0, 16)], acc.at[pl.ds(9984, 16)])

    plsc.subcore_barrier()

    def fire(j, carry):
        pltpu.async_copy(
            ones, acc.at[dstv.at[pl.ds(j * BUF_E, BUF_E)]], dsem, add=True
        )
        return carry

    lax.fori_loop(0, ne // BUF_E, fire, 0)

    def drain(j, carry):
        pltpu.make_async_copy(
            ones, acc.at[dstv.at[pl.ds(0, BUF_E)]], dsem
        ).wait()
        return carry

    lax.fori_loop(0, ne // BUF_E, drain, 0)
    plsc.subcore_barrier()
    for i in range(6):
        sl = pl.ds(base + ZROWS * i, ZROWS)
        pltpu.sync_copy(acc.at[sl], zbuf)
        pltpu.sync_copy(zbuf, out_hbm.at[pl.ds(c * N_NODES + base + ZROWS * i, ZROWS)])

    @pl.when(s == 15)
    def _():
        pltpu.sync_copy(acc.at[pl.ds(9984, 16)], zbuf.at[pl.ds(0, 16)])
        pltpu.sync_copy(zbuf.at[pl.ds(0, 16)], out_hbm.at[pl.ds(c * N_NODES + 9984, 16)])


@functools.partial(
    pl.kernel,
    mesh=_mesh(),
    compiler_params=_SC_PARAMS,
    out_type=jax.ShapeDtypeStruct((NC, N_NODES, 64), jnp.float32),
    scratch_types=[
        pltpu.VMEM((SLAB_ROWS // 5 * CHUNK,), jnp.int32),
        pltpu.VMEM((SLAB_ROWS // 5 * CHUNK,), jnp.int32),
        pltpu.VMEM((SLAB_ROWS // 5 * CHUNK,), jnp.int32),
        pltpu.VMEM((SLAB_ROWS // 5 * CHUNK,), jnp.int32),
        pltpu.VMEM((BUF_E, 64), jnp.float32),
        pltpu.VMEM((BUF_E, 64), jnp.float32),
        pltpu.VMEM((ZROWS, 64), jnp.float32),
        pltpu.VMEM_SHARED((N_NODES, 64), jnp.float32),
        pltpu.SemaphoreType.DMA,
        pltpu.SemaphoreType.DMA,
        pltpu.SemaphoreType.DMA,
        pltpu.SemaphoreType.DMA,
        pltpu.SemaphoreType.DMA,
        pltpu.SemaphoreType.DMA,
    ],
)
def _smooth_full(src_hbm, dst_hbm, g_hbm, out_hbm, isrc0, isrc1, idst0, idst1,
                 buf0, buf1, zbuf, acc, is0, is1, gs0, gs1, ss0, ss1):
    c = lax.axis_index("c")
    s = lax.axis_index("s")
    _zero_vmem(zbuf, ZROWS, 64)
    _zero_acc(acc, zbuf, s)
    plsc.subcore_barrier()
    _edge_pipeline(
        src_hbm.at[s], dst_hbm.at[s], g_hbm.at[c], acc,
        (isrc0, isrc1), (idst0, idst1), (is0, is1),
        (buf0, buf1), (gs0, gs1), (ss0, ss1),
        0, SLAB_ROWS,
    )
    plsc.subcore_barrier()
    _copy_out(acc, buf0, out_hbm.at[c], s)


@functools.partial(
    pl.kernel,
    mesh=_mesh(),
    compiler_params=_SC_PARAMS,
    out_type=jax.ShapeDtypeStruct((NC, N_NODES, 64), jnp.float32),
    scratch_types=[
        pltpu.VMEM((SLAB_ROWS // NC // 5 * CHUNK,), jnp.int32),
        pltpu.VMEM((SLAB_ROWS // NC // 5 * CHUNK,), jnp.int32),
        pltpu.VMEM((SLAB_ROWS // NC // 5 * CHUNK,), jnp.int32),
        pltpu.VMEM((SLAB_ROWS // NC // 5 * CHUNK,), jnp.int32),
        pltpu.VMEM((BUF_E, 64), jnp.float32),
        pltpu.VMEM((BUF_E, 64), jnp.float32),
        pltpu.VMEM((ZROWS, 64), jnp.float32),
        pltpu.VMEM_SHARED((N_NODES, 64), jnp.float32),
        pltpu.SemaphoreType.DMA,
        pltpu.SemaphoreType.DMA,
        pltpu.SemaphoreType.DMA,
        pltpu.SemaphoreType.DMA,
        pltpu.SemaphoreType.DMA,
        pltpu.SemaphoreType.DMA,
    ],
)
def _smooth_half(src_hbm, dst_hbm, g_hbm, out_hbm, isrc0, isrc1, idst0, idst1,
                 buf0, buf1, zbuf, acc, is0, is1, gs0, gs1, ss0, ss1):
    c = lax.axis_index("c")
    s = lax.axis_index("s")
    nch = SLAB_ROWS // NC
    _zero_vmem(zbuf, ZROWS, 64)
    _zero_acc(acc, zbuf, s)
    plsc.subcore_barrier()
    _edge_pipeline(
        src_hbm.at[s], dst_hbm.at[s], g_hbm, acc,
        (isrc0, isrc1), (idst0, idst1), (is0, is1),
        (buf0, buf1), (gs0, gs1), (ss0, ss1),
        c * nch, nch,
    )
    plsc.subcore_barrier()
    _copy_out(acc, buf0, out_hbm.at[c], s)


def _tc_layer0(deg, x, w0, b0):
    def body(deg_ref, x_ref, w_ref, b_ref, norm_ref, g0_ref):
        dsum = deg_ref[0, :] + deg_ref[1, :] + 1.0
        norm = lax.rsqrt(dsum)
        norm_ref[...] = norm[:, None]
        h = jnp.dot(x_ref[...], w_ref[...], preferred_element_type=jnp.float32)
        h = h + b_ref[...]
        g0 = h * norm[:, None]
        g0_ref[0] = g0[:, :64]
        g0_ref[1] = g0[:, 64:]

    return pl.pallas_call(
        body,
        out_shape=(
            jax.ShapeDtypeStruct((N_NODES, 1), jnp.float32),
            jax.ShapeDtypeStruct((NC, N_NODES, 64), jnp.float32),
        ),
    )(deg, x, w0, b0)


def _tc_layer1(p, g0, norm, w1, b1):
    def body(p_ref, g0_ref, norm_ref, w_ref, b_ref, g1_ref):
        sm = jnp.concatenate(
            [p_ref[0] + g0_ref[0], p_ref[1] + g0_ref[1]], axis=1
        ) * norm_ref[...]
        h1 = jnp.maximum(sm, 0.0)
        o = jnp.dot(h1, w_ref[...], preferred_element_type=jnp.float32)
        o = o + b_ref[...]
        g1_ref[...] = o * norm_ref[...]

    return pl.pallas_call(
        body,
        out_shape=jax.ShapeDtypeStruct((N_NODES, w1.shape[1]), jnp.float32),
    )(p, g0, norm, w1, b1)


def _tc_final(q, g1, norm):
    def body(q_ref, g1_ref, norm_ref, o_ref):
        o_ref[...] = (q_ref[0] + q_ref[1] + g1_ref[...]) * norm_ref[...]

    return pl.pallas_call(
        body,
        out_shape=jax.ShapeDtypeStruct(g1.shape, jnp.float32),
    )(q, g1, norm)


def kernel(X, edge_index, W0, b0, W1, b1):
    src = edge_index[0].astype(jnp.int32).reshape(NS, SLAB_ROWS * CHUNK)
    dst = edge_index[1].astype(jnp.int32).reshape(NS, SLAB_ROWS * CHUNK)
    deg = _deg_kernel(dst).reshape(NC, N_NODES)
    norm, g0 = _tc_layer0(deg, X, W0, b0.reshape(1, -1))
    p = _smooth_full(src, dst, g0)
    g1 = _tc_layer1(p, g0, norm, W1, b1.reshape(1, -1))
    q = _smooth_half(src, dst, g1)
    return _tc_final(q, g1, norm)
